# baseline (speedup 1.0000x reference)
import dataclasses

import jax
import jax.numpy as jnp
from jax import lax
from jax.experimental import pallas as pl
from jax.experimental.pallas import tpu as pltpu
from jax.experimental.pallas import tpu_sc as plsc

N = 10000
DIM_O = 64
H = 128
NP = 10240
E = 320000
EP = 327680
TRASH = N
L = 16
NTILES = 16
STRIPE = NP // NTILES
REG = EP // 32
RCAP = REG + 256
RROWS = RCAP // 128

_HIGH = lax.Precision.HIGHEST

_mesh = plsc.VectorSubcoreMesh(core_axis_name="c", subcore_axis_name="s")

_sc_params = pltpu.CompilerParams()
if "needs_layout_passes" in pltpu.CompilerParams.__dataclass_fields__:
    _sc_params = dataclasses.replace(_sc_params, needs_layout_passes=False)


def _zeros16():
    return jnp.zeros((L,), jnp.int32)


def _ones16():
    return jnp.ones((L,), jnp.int32)


def _m1_body(src_hbm, dst_hbm, cm_hbm,
             srcc_hbm, dstc_hbm, cnt_hbm, cpart_hbm,
             cm_v, srcb, dstb, sstage, dstage, cnt_v, red_v, part_st, cnt_st,
             shared_cnt):
    cid = lax.axis_index("c")
    sid = lax.axis_index("s")
    r = cid * NTILES + sid

    pltpu.sync_copy(cm_hbm, cm_v)
    pltpu.sync_copy(src_hbm.at[pl.ds(r * REG, REG)], srcb)
    pltpu.sync_copy(dst_hbm.at[pl.ds(r * REG, REG)], dstb)

    @pl.loop(0, NP, step=L)
    def _(i):
        cnt_v[pl.ds(i, L)] = _zeros16()

    trash16 = jnp.full((L,), TRASH, jnp.int32)

    @pl.loop(0, REG, step=L, init_carry=jnp.int32(0))
    def comp(i, off):
        s16 = srcb[pl.ds(i, L)]
        d16 = dstb[pl.ds(i, L)]
        cs = plsc.load_gather(cm_v, [s16])
        cd = plsc.load_gather(cm_v, [d16])
        m = jnp.logical_and(cs > 0, cd == 0)
        plsc.store_compressed(sstage.at[pl.ds(off, L)], s16, mask=m)
        plsc.store_compressed(dstage.at[pl.ds(off, L)], d16, mask=m)
        plsc.addupdate_scatter(cnt_v, [d16], _ones16(), mask=m)
        return off + jnp.sum(m.astype(jnp.int32))

    off = comp

    iota16 = lax.iota(jnp.int32, L)

    @pl.loop(0, 256, step=L)
    def _(j):
        tv = TRASH + jnp.bitwise_and(iota16 + j, 127)
        sstage[pl.ds(off + j, L)] = tv
        dstage[pl.ds(off + j, L)] = tv

    rows_cnt = lax.shift_right_logical(off + 127, 7)
    cnt_st[...] = jnp.broadcast_to(rows_cnt, (L,)).astype(jnp.int32)
    pltpu.sync_copy(sstage, srcc_hbm.at[pl.ds(r * RCAP, RCAP)])
    pltpu.sync_copy(dstage, dstc_hbm.at[pl.ds(r * RCAP, RCAP)])
    pltpu.sync_copy(cnt_st, cnt_hbm.at[r])

    pltpu.sync_copy(cnt_v, shared_cnt.at[sid])
    plsc.subcore_barrier()

    nbase = sid * STRIPE
    for t in range(NTILES):
        pltpu.sync_copy(shared_cnt.at[t, pl.ds(nbase, STRIPE)],
                        red_v.at[pl.ds(t * STRIPE, STRIPE)])

    @pl.loop(0, STRIPE, step=L)
    def _(k):
        tot = red_v[pl.ds(k, L)]
        for t in range(1, NTILES):
            tot = tot + red_v[pl.ds(t * STRIPE + k, L)]
        part_st[pl.ds(k, L)] = tot

    pltpu.sync_copy(part_st, cpart_hbm.at[cid].at[pl.ds(nbase, STRIPE)])


_m1_call = pl.kernel(
    _m1_body,
    out_type=(
        jax.ShapeDtypeStruct((32 * RCAP,), jnp.int32),
        jax.ShapeDtypeStruct((32 * RCAP,), jnp.int32),
        jax.ShapeDtypeStruct((32, L), jnp.int32),
        jax.ShapeDtypeStruct((2, NP), jnp.int32),
    ),
    mesh=_mesh,
    scratch_types=[
        pltpu.VMEM((NP,), jnp.int32),
        pltpu.VMEM((REG,), jnp.int32),
        pltpu.VMEM((REG,), jnp.int32),
        pltpu.VMEM((RCAP,), jnp.int32),
        pltpu.VMEM((RCAP,), jnp.int32),
        pltpu.VMEM((NP,), jnp.int32),
        pltpu.VMEM((NTILES * STRIPE,), jnp.int32),
        pltpu.VMEM((STRIPE,), jnp.int32),
        pltpu.VMEM((L,), jnp.int32),
        pltpu.VMEM_SHARED((NTILES, NP), jnp.int32),
    ],
    compiler_params=_sc_params,
    name="sc_mask_step1",
)


def _m2_body(src_hbm, dst_hbm, cm_hbm, cpart1_hbm,
             srcc_hbm, dstc_hbm, cnt_hbm, cpart_hbm,
             ns_v, mt_v, srcb, dstb, sstage, dstage, cnt_v, red_v, part_st,
             cnt_st, shared_cnt):
    cid = lax.axis_index("c")
    sid = lax.axis_index("s")
    r = cid * NTILES + sid

    pltpu.sync_copy(cpart1_hbm.at[0], ns_v)
    pltpu.sync_copy(cpart1_hbm.at[1], mt_v)
    pltpu.sync_copy(cm_hbm, cnt_v)

    @pl.loop(0, NP, step=L)
    def _(i):
        deg = ns_v[pl.ds(i, L)] + mt_v[pl.ds(i, L)]
        cm16 = cnt_v[pl.ds(i, L)]
        one = _ones16()
        zero = _zeros16()
        ns_v[pl.ds(i, L)] = jnp.where(deg > 0, one, zero)
        mt_v[pl.ds(i, L)] = jnp.where(
            jnp.logical_and(deg == 0, cm16 == 0), one, zero)

    pltpu.sync_copy(src_hbm.at[pl.ds(r * REG, REG)], srcb)
    pltpu.sync_copy(dst_hbm.at[pl.ds(r * REG, REG)], dstb)

    @pl.loop(0, NP, step=L)
    def _(i):
        cnt_v[pl.ds(i, L)] = _zeros16()

    trash16 = jnp.full((L,), TRASH, jnp.int32)

    @pl.loop(0, REG, step=L, init_carry=jnp.int32(0))
    def comp(i, off):
        s16 = srcb[pl.ds(i, L)]
        d16 = dstb[pl.ds(i, L)]
        ms = plsc.load_gather(ns_v, [s16])
        mt = plsc.load_gather(mt_v, [d16])
        m = jnp.logical_and(ms > 0, mt > 0)
        plsc.store_compressed(sstage.at[pl.ds(off, L)], s16, mask=m)
        plsc.store_compressed(dstage.at[pl.ds(off, L)], d16, mask=m)
        plsc.addupdate_scatter(cnt_v, [d16], _ones16(), mask=m)
        return off + jnp.sum(m.astype(jnp.int32))

    off = comp

    iota16 = lax.iota(jnp.int32, L)

    @pl.loop(0, 256, step=L)
    def _(j):
        tv = TRASH + jnp.bitwise_and(iota16 + j, 127)
        sstage[pl.ds(off + j, L)] = tv
        dstage[pl.ds(off + j, L)] = tv

    rows_cnt = lax.shift_right_logical(off + 127, 7)
    cnt_st[...] = jnp.broadcast_to(rows_cnt, (L,)).astype(jnp.int32)
    pltpu.sync_copy(sstage, srcc_hbm.at[pl.ds(r * RCAP, RCAP)])
    pltpu.sync_copy(dstage, dstc_hbm.at[pl.ds(r * RCAP, RCAP)])
    pltpu.sync_copy(cnt_st, cnt_hbm.at[r])

    pltpu.sync_copy(cnt_v, shared_cnt.at[sid])
    plsc.subcore_barrier()

    nbase = sid * STRIPE
    for t in range(NTILES):
        pltpu.sync_copy(shared_cnt.at[t, pl.ds(nbase, STRIPE)],
                        red_v.at[pl.ds(t * STRIPE, STRIPE)])

    @pl.loop(0, STRIPE, step=L)
    def _(k):
        tot = red_v[pl.ds(k, L)]
        for t in range(1, NTILES):
            tot = tot + red_v[pl.ds(t * STRIPE + k, L)]
        part_st[pl.ds(k, L)] = tot

    pltpu.sync_copy(part_st, cpart_hbm.at[cid].at[pl.ds(nbase, STRIPE)])


_m2_call = pl.kernel(
    _m2_body,
    out_type=(
        jax.ShapeDtypeStruct((32 * RCAP,), jnp.int32),
        jax.ShapeDtypeStruct((32 * RCAP,), jnp.int32),
        jax.ShapeDtypeStruct((32, L), jnp.int32),
        jax.ShapeDtypeStruct((2, NP), jnp.int32),
    ),
    mesh=_mesh,
    scratch_types=[
        pltpu.VMEM((NP,), jnp.int32),
        pltpu.VMEM((NP,), jnp.int32),
        pltpu.VMEM((REG,), jnp.int32),
        pltpu.VMEM((REG,), jnp.int32),
        pltpu.VMEM((RCAP,), jnp.int32),
        pltpu.VMEM((RCAP,), jnp.int32),
        pltpu.VMEM((NP,), jnp.int32),
        pltpu.VMEM((NTILES * STRIPE,), jnp.int32),
        pltpu.VMEM((STRIPE,), jnp.int32),
        pltpu.VMEM((L,), jnp.int32),
        pltpu.VMEM_SHARED((NTILES, NP), jnp.int32),
    ],
    compiler_params=_sc_params,
    name="sc_mask_step2",
)


def _agg_body(p_hbm, srcc_hbm, dstc_hbm, cnt_hbm, zer_hbm, out_hbm,
              srcb, didx, cntb, rows0, rows1, zbuf, acc,
              sem0, sem1, semd0, semd1):
    cid = lax.axis_index("c")
    sid = lax.axis_index("s")
    r = cid * NTILES + sid
    nbase = sid * STRIPE

    pltpu.sync_copy(zer_hbm, zbuf)

    @pl.loop(0, STRIPE, step=32)
    def _(k):
        pltpu.sync_copy(zbuf, acc.at[pl.ds(nbase + k, 32)])

    pltpu.sync_copy(cnt_hbm.at[r], cntb)
    pltpu.sync_copy(srcc_hbm.at[pl.ds(r * RCAP, RCAP)], srcb)
    rows_cnt = jnp.max(cntb[...])
    plsc.subcore_barrier()

    even_cnt = jnp.bitwise_and(rows_cnt, jnp.int32(-2))

    @pl.loop(0, even_cnt, step=2)
    def _(c):
        dd0 = pltpu.async_copy(
            dstc_hbm.at[pl.ds(r * RCAP + c * 128, 128)], didx.at[0], semd0)
        g0 = pltpu.async_copy(
            p_hbm.at[srcb.at[pl.ds(c * 128, 128)]], rows0, sem0)
        dd1 = pltpu.async_copy(
            dstc_hbm.at[pl.ds(r * RCAP + c * 128 + 128, 128)], didx.at[1],
            semd1)
        g1 = pltpu.async_copy(
            p_hbm.at[srcb.at[pl.ds(c * 128 + 128, 128)]], rows1, sem1)
        dd0.wait()
        g0.wait()
        pltpu.sync_copy(rows0, acc.at[didx.at[0]], add=True)
        dd1.wait()
        g1.wait()
        pltpu.sync_copy(rows1, acc.at[didx.at[1]], add=True)

    @pl.when(even_cnt != rows_cnt)
    def _():
        c = even_cnt
        dd0 = pltpu.async_copy(
            dstc_hbm.at[pl.ds(r * RCAP + c * 128, 128)], didx.at[0], semd0)
        g0 = pltpu.async_copy(
            p_hbm.at[srcb.at[pl.ds(c * 128, 128)]], rows0, sem0)
        dd0.wait()
        g0.wait()
        pltpu.sync_copy(rows0, acc.at[didx.at[0]], add=True)

    plsc.subcore_barrier()
    pltpu.sync_copy(acc.at[pl.ds(nbase, STRIPE)],
                    out_hbm.at[cid].at[pl.ds(nbase, STRIPE)])


_agg_call = pl.kernel(
    _agg_body,
    out_type=jax.ShapeDtypeStruct((2, NP, H), jnp.float32),
    mesh=_mesh,
    scratch_types=[
        pltpu.VMEM((RCAP,), jnp.int32),
        pltpu.VMEM((2, 128), jnp.int32),
        pltpu.VMEM((L,), jnp.int32),
        pltpu.VMEM((128, H), jnp.float32),
        pltpu.VMEM((128, H), jnp.float32),
        pltpu.VMEM((32, H), jnp.float32),
        pltpu.VMEM_SHARED((NP, H), jnp.float32),
        pltpu.SemaphoreType.DMA,
        pltpu.SemaphoreType.DMA,
        pltpu.SemaphoreType.DMA,
        pltpu.SemaphoreType.DMA,
    ],
    compiler_params=_sc_params,
    name="sc_gather_segsum",
)


B = 2560
GP = NP // B


def _pre_body(x_ref, cm_ref, wio_ref, wiu_ref, wo1_ref, wu1_ref,
              xo_ref, xu_ref, p1_ref, s1_ref, s0_ref, sc_ref):
    i = pl.program_id(0)
    x = x_ref[...]
    cm = cm_ref[...]
    xo = lax.dot(x[:, :DIM_O], wio_ref[...], precision=_HIGH)
    xu = lax.dot(x[:, DIM_O:], wiu_ref[...], precision=_HIGH)
    xo_ref[...] = xo
    xu_ref[...] = xu
    p1_ref[...] = (lax.dot(xo, wo1_ref[...], precision=_HIGH)
                   + lax.dot(xu, wu1_ref[...], precision=_HIGH))

    @pl.when(i == 0)
    def _():
        s1_ref[...] = jnp.zeros_like(s1_ref)
        s0_ref[...] = jnp.zeros_like(s0_ref)
        sc_ref[...] = jnp.zeros_like(sc_ref)

    s1_ref[...] += jnp.sum(xo * cm, axis=0, keepdims=True)
    s0_ref[...] += jnp.sum(xo * (1.0 - cm), axis=0, keepdims=True)
    sc_ref[...] += jnp.sum(cm, keepdims=True)


_pre_call = pl.pallas_call(
    _pre_body,
    grid=(GP,),
    in_specs=[
        pl.BlockSpec((B, 2 * DIM_O), lambda i: (i, 0)),
        pl.BlockSpec((B, 1), lambda i: (i, 0)),
        pl.BlockSpec((DIM_O, H), lambda i: (0, 0)),
        pl.BlockSpec((DIM_O, H), lambda i: (0, 0)),
        pl.BlockSpec((H, H), lambda i: (0, 0)),
        pl.BlockSpec((H, H), lambda i: (0, 0)),
    ],
    out_specs=[
        pl.BlockSpec((B, H), lambda i: (i, 0)),
        pl.BlockSpec((B, H), lambda i: (i, 0)),
        pl.BlockSpec((B, H), lambda i: (i, 0)),
        pl.BlockSpec((1, H), lambda i: (0, 0)),
        pl.BlockSpec((1, H), lambda i: (0, 0)),
        pl.BlockSpec((1, 1), lambda i: (0, 0)),
    ],
    out_shape=(
        jax.ShapeDtypeStruct((NP, H), jnp.float32),
        jax.ShapeDtypeStruct((NP, H), jnp.float32),
        jax.ShapeDtypeStruct((NP, H), jnp.float32),
        jax.ShapeDtypeStruct((1, H), jnp.float32),
        jax.ShapeDtypeStruct((1, H), jnp.float32),
        jax.ShapeDtypeStruct((1, 1), jnp.float32),
    ),
)


def _mid_body(xo_ref, xu_ref, part_ref, cnt_ref, s1_ref, s0_ref, sc_ref,
              wd_ref, wo2_ref, wu2_ref, xuh_ref, p2_ref):
    nc = jnp.maximum(sc_ref[0, 0], 1.0)
    nnc = jnp.maximum(jnp.float32(N) - sc_ref[0, 0], 1.0)
    delta = s1_ref[...] / nc - s0_ref[...] / nnc
    add = lax.dot(delta, wd_ref[...], precision=_HIGH)
    deg = (cnt_ref[0] + cnt_ref[1]).astype(jnp.float32)
    agg = (part_ref[0] + part_ref[1]) / jnp.maximum(deg, 1.0)
    xuh = xu_ref[...] + agg + add
    xuh_ref[...] = xuh
    p2_ref[...] = (lax.dot(xo_ref[...], wo2_ref[...], precision=_HIGH)
                   + lax.dot(xuh, wu2_ref[...], precision=_HIGH))


_mid_call = pl.pallas_call(
    _mid_body,
    grid=(GP,),
    in_specs=[
        pl.BlockSpec((B, H), lambda i: (i, 0)),
        pl.BlockSpec((B, H), lambda i: (i, 0)),
        pl.BlockSpec((2, B, H), lambda i: (0, i, 0)),
        pl.BlockSpec((2, B, 1), lambda i: (0, i, 0)),
        pl.BlockSpec((1, H), lambda i: (0, 0)),
        pl.BlockSpec((1, H), lambda i: (0, 0)),
        pl.BlockSpec((1, 1), lambda i: (0, 0)),
        pl.BlockSpec((H, H), lambda i: (0, 0)),
        pl.BlockSpec((H, H), lambda i: (0, 0)),
        pl.BlockSpec((H, H), lambda i: (0, 0)),
    ],
    out_specs=[
        pl.BlockSpec((B, H), lambda i: (i, 0)),
        pl.BlockSpec((B, H), lambda i: (i, 0)),
    ],
    out_shape=(
        jax.ShapeDtypeStruct((NP, H), jnp.float32),
        jax.ShapeDtypeStruct((NP, H), jnp.float32),
    ),
)


def _post_body(xo_ref, xu_ref, xuh_ref, part_ref, cnt_ref, cm_ref, out_ref):
    deg = (cnt_ref[0] + cnt_ref[1]).astype(jnp.float32)
    final_u = xuh_ref[...] + (part_ref[0] + part_ref[1]) / jnp.maximum(deg, 1.0)
    final_u = jnp.where(cm_ref[...] > 0.0, xu_ref[...], final_u)
    out_ref[...] = jnp.concatenate([xo_ref[...], final_u], axis=1)


_post_call = pl.pallas_call(
    _post_body,
    grid=(GP,),
    in_specs=[
        pl.BlockSpec((B, H), lambda i: (i, 0)),
        pl.BlockSpec((B, H), lambda i: (i, 0)),
        pl.BlockSpec((B, H), lambda i: (i, 0)),
        pl.BlockSpec((2, B, H), lambda i: (0, i, 0)),
        pl.BlockSpec((2, B, 1), lambda i: (0, i, 0)),
        pl.BlockSpec((B, 1), lambda i: (i, 0)),
    ],
    out_specs=pl.BlockSpec((B, 2 * H), lambda i: (i, 0)),
    out_shape=jax.ShapeDtypeStruct((NP, 2 * H), jnp.float32),
)


@jax.jit
def _impl(x, edge_index, central_mask,
          W_in_o, W_in_u, W_o1, W_u1, W_delta, W_o2, W_u2):
    xp = jnp.zeros((NP, DIM_O + DIM_O), x.dtype).at[:N].set(x)
    cm_i = jnp.zeros((NP,), jnp.int32).at[:N].set(central_mask.astype(jnp.int32))
    cm_f = cm_i.astype(jnp.float32).reshape(NP, 1)
    pad = jnp.full((EP - E,), TRASH, jnp.int32)
    src1 = jnp.concatenate([edge_index[0], pad])
    dst1 = jnp.concatenate([edge_index[1], pad])
    zer = jnp.zeros((32, H), jnp.float32)

    sc1, dc1, cnt1, cp1 = _m1_call(src1, dst1, cm_i)
    xo, xu, p1, s1, s0, sc = _pre_call(xp, cm_f, W_in_o, W_in_u, W_o1, W_u1)
    part1 = _agg_call(p1, sc1, dc1, cnt1, zer)
    sc2, dc2, cnt2, cp2 = _m2_call(src1, dst1, cm_i, cp1)
    xuh, p2 = _mid_call(xo, xu, part1, cp1.reshape(2, NP, 1), s1, s0, sc,
                        W_delta, W_o2, W_u2)
    part2 = _agg_call(p2, sc2, dc2, cnt2, zer)
    out = _post_call(xo, xu, xuh, part2, cp2.reshape(2, NP, 1), cm_f)
    return out[:N]


def kernel(x, edge_index, central_mask,
           W_in_o, W_in_u, W_o1, W_u1, W_delta, W_o2, W_u2):
    return _impl(x, edge_index, central_mask,
                 W_in_o, W_in_u, W_o1, W_u1, W_delta, W_o2, W_u2)


# speedup vs baseline: 1.0512x; 1.0512x over previous
import dataclasses

import jax
import jax.numpy as jnp
from jax import lax
from jax.experimental import pallas as pl
from jax.experimental.pallas import tpu as pltpu
from jax.experimental.pallas import tpu_sc as plsc

N = 10000
DIM_O = 64
H = 128
NP = 10240
E = 320000
EP = 327680
TRASH = N
L = 16
NTILES = 16
STRIPE = NP // NTILES
REG = EP // 32
RCAP = REG + 256
RROWS = RCAP // 128

_HIGH = lax.Precision.HIGHEST

_mesh = plsc.VectorSubcoreMesh(core_axis_name="c", subcore_axis_name="s")

_sc_params = pltpu.CompilerParams()
if "needs_layout_passes" in pltpu.CompilerParams.__dataclass_fields__:
    _sc_params = dataclasses.replace(_sc_params, needs_layout_passes=False)


def _zeros16():
    return jnp.zeros((L,), jnp.int32)


def _ones16():
    return jnp.ones((L,), jnp.int32)


def _m1_body(src_hbm, dst_hbm, cm_hbm,
             srcc_hbm, dstc_hbm, cnt_hbm, cpart_hbm,
             cm_v, srcb, dstb, sstage, dstage, cnt_v, red_v, part_st, cnt_st,
             shared_cnt):
    cid = lax.axis_index("c")
    sid = lax.axis_index("s")
    r = cid * NTILES + sid

    pltpu.sync_copy(cm_hbm, cm_v)
    pltpu.sync_copy(src_hbm.at[pl.ds(r * REG, REG)], srcb)
    pltpu.sync_copy(dst_hbm.at[pl.ds(r * REG, REG)], dstb)

    @pl.loop(0, NP, step=L)
    def _(i):
        cnt_v[pl.ds(i, L)] = _zeros16()

    trash16 = jnp.full((L,), TRASH, jnp.int32)

    @pl.loop(0, REG, step=L, init_carry=jnp.int32(0))
    def comp(i, off):
        s16 = srcb[pl.ds(i, L)]
        d16 = dstb[pl.ds(i, L)]
        cs = plsc.load_gather(cm_v, [s16])
        cd = plsc.load_gather(cm_v, [d16])
        m = jnp.logical_and(cs > 0, cd == 0)
        plsc.store_compressed(sstage.at[pl.ds(off, L)], s16, mask=m)
        plsc.store_compressed(dstage.at[pl.ds(off, L)], d16, mask=m)
        plsc.addupdate_scatter(cnt_v, [d16], _ones16(), mask=m)
        return off + jnp.sum(m.astype(jnp.int32))

    off = comp

    iota16 = lax.iota(jnp.int32, L)

    @pl.loop(0, 256, step=L)
    def _(j):
        tv = TRASH + jnp.bitwise_and(iota16 + j, 127)
        sstage[pl.ds(off + j, L)] = tv
        dstage[pl.ds(off + j, L)] = tv

    rows_cnt = lax.shift_right_logical(off + 127, 7)
    cnt_st[...] = jnp.broadcast_to(rows_cnt, (L,)).astype(jnp.int32)
    pltpu.sync_copy(sstage, srcc_hbm.at[pl.ds(r * RCAP, RCAP)])
    pltpu.sync_copy(dstage, dstc_hbm.at[pl.ds(r * RCAP, RCAP)])
    pltpu.sync_copy(cnt_st, cnt_hbm.at[r])

    pltpu.sync_copy(cnt_v, shared_cnt.at[sid])
    plsc.subcore_barrier()

    nbase = sid * STRIPE
    for t in range(NTILES):
        pltpu.sync_copy(shared_cnt.at[t, pl.ds(nbase, STRIPE)],
                        red_v.at[pl.ds(t * STRIPE, STRIPE)])

    @pl.loop(0, STRIPE, step=L)
    def _(k):
        tot = red_v[pl.ds(k, L)]
        for t in range(1, NTILES):
            tot = tot + red_v[pl.ds(t * STRIPE + k, L)]
        part_st[pl.ds(k, L)] = tot

    pltpu.sync_copy(part_st, cpart_hbm.at[cid].at[pl.ds(nbase, STRIPE)])


_m1_call = pl.kernel(
    _m1_body,
    out_type=(
        jax.ShapeDtypeStruct((32 * RCAP,), jnp.int32),
        jax.ShapeDtypeStruct((32 * RCAP,), jnp.int32),
        jax.ShapeDtypeStruct((32, L), jnp.int32),
        jax.ShapeDtypeStruct((2, NP), jnp.int32),
    ),
    mesh=_mesh,
    scratch_types=[
        pltpu.VMEM((NP,), jnp.int32),
        pltpu.VMEM((REG,), jnp.int32),
        pltpu.VMEM((REG,), jnp.int32),
        pltpu.VMEM((RCAP,), jnp.int32),
        pltpu.VMEM((RCAP,), jnp.int32),
        pltpu.VMEM((NP,), jnp.int32),
        pltpu.VMEM((NTILES * STRIPE,), jnp.int32),
        pltpu.VMEM((STRIPE,), jnp.int32),
        pltpu.VMEM((L,), jnp.int32),
        pltpu.VMEM_SHARED((NTILES, NP), jnp.int32),
    ],
    compiler_params=_sc_params,
    name="sc_mask_step1",
)


def _m2_body(src_hbm, dst_hbm, ns_hbm, mt2_hbm,
             srcc_hbm, dstc_hbm, cnt_hbm, cpart_hbm,
             ns_v, mt_v, srcb, dstb, sstage, dstage, cnt_v, red_v, part_st,
             cnt_st, shared_cnt):
    cid = lax.axis_index("c")
    sid = lax.axis_index("s")
    r = cid * NTILES + sid

    pltpu.sync_copy(ns_hbm, ns_v)
    pltpu.sync_copy(mt2_hbm, mt_v)
    pltpu.sync_copy(src_hbm.at[pl.ds(r * REG, REG)], srcb)
    pltpu.sync_copy(dst_hbm.at[pl.ds(r * REG, REG)], dstb)

    @pl.loop(0, NP, step=L)
    def _(i):
        cnt_v[pl.ds(i, L)] = _zeros16()

    trash16 = jnp.full((L,), TRASH, jnp.int32)

    @pl.loop(0, REG, step=L, init_carry=jnp.int32(0))
    def comp(i, off):
        s16 = srcb[pl.ds(i, L)]
        d16 = dstb[pl.ds(i, L)]
        ms = plsc.load_gather(ns_v, [s16])
        mt = plsc.load_gather(mt_v, [d16])
        m = jnp.logical_and(ms > 0, mt > 0)
        plsc.store_compressed(sstage.at[pl.ds(off, L)], s16, mask=m)
        plsc.store_compressed(dstage.at[pl.ds(off, L)], d16, mask=m)
        plsc.addupdate_scatter(cnt_v, [d16], _ones16(), mask=m)
        return off + jnp.sum(m.astype(jnp.int32))

    off = comp

    iota16 = lax.iota(jnp.int32, L)

    @pl.loop(0, 256, step=L)
    def _(j):
        tv = TRASH + jnp.bitwise_and(iota16 + j, 127)
        sstage[pl.ds(off + j, L)] = tv
        dstage[pl.ds(off + j, L)] = tv

    rows_cnt = lax.shift_right_logical(off + 127, 7)
    cnt_st[...] = jnp.broadcast_to(rows_cnt, (L,)).astype(jnp.int32)
    pltpu.sync_copy(sstage, srcc_hbm.at[pl.ds(r * RCAP, RCAP)])
    pltpu.sync_copy(dstage, dstc_hbm.at[pl.ds(r * RCAP, RCAP)])
    pltpu.sync_copy(cnt_st, cnt_hbm.at[r])

    pltpu.sync_copy(cnt_v, shared_cnt.at[sid])
    plsc.subcore_barrier()

    nbase = sid * STRIPE
    for t in range(NTILES):
        pltpu.sync_copy(shared_cnt.at[t, pl.ds(nbase, STRIPE)],
                        red_v.at[pl.ds(t * STRIPE, STRIPE)])

    @pl.loop(0, STRIPE, step=L)
    def _(k):
        tot = red_v[pl.ds(k, L)]
        for t in range(1, NTILES):
            tot = tot + red_v[pl.ds(t * STRIPE + k, L)]
        part_st[pl.ds(k, L)] = tot

    pltpu.sync_copy(part_st, cpart_hbm.at[cid].at[pl.ds(nbase, STRIPE)])


_m2_call = pl.kernel(
    _m2_body,
    out_type=(
        jax.ShapeDtypeStruct((32 * RCAP,), jnp.int32),
        jax.ShapeDtypeStruct((32 * RCAP,), jnp.int32),
        jax.ShapeDtypeStruct((32, L), jnp.int32),
        jax.ShapeDtypeStruct((2, NP), jnp.int32),
    ),
    mesh=_mesh,
    scratch_types=[
        pltpu.VMEM((NP,), jnp.int32),
        pltpu.VMEM((NP,), jnp.int32),
        pltpu.VMEM((REG,), jnp.int32),
        pltpu.VMEM((REG,), jnp.int32),
        pltpu.VMEM((RCAP,), jnp.int32),
        pltpu.VMEM((RCAP,), jnp.int32),
        pltpu.VMEM((NP,), jnp.int32),
        pltpu.VMEM((NTILES * STRIPE,), jnp.int32),
        pltpu.VMEM((STRIPE,), jnp.int32),
        pltpu.VMEM((L,), jnp.int32),
        pltpu.VMEM_SHARED((NTILES, NP), jnp.int32),
    ],
    compiler_params=_sc_params,
    name="sc_mask_step2",
)


def _agg_body(p_hbm, srcc_hbm, dstc_hbm, cnt_hbm, zer_hbm, out_hbm,
              srcb, didx, cntb, rows0, rows1, zbuf, acc,
              sem0, sem1, semd0, semd1):
    cid = lax.axis_index("c")
    sid = lax.axis_index("s")
    r = cid * NTILES + sid
    nbase = sid * STRIPE

    pltpu.sync_copy(zer_hbm, zbuf)

    @pl.loop(0, STRIPE, step=32)
    def _(k):
        pltpu.sync_copy(zbuf, acc.at[pl.ds(nbase + k, 32)])

    pltpu.sync_copy(cnt_hbm.at[r], cntb)
    pltpu.sync_copy(srcc_hbm.at[pl.ds(r * RCAP, RCAP)], srcb)
    rows_cnt = jnp.max(cntb[...])
    plsc.subcore_barrier()

    even_cnt = jnp.bitwise_and(rows_cnt, jnp.int32(-2))

    @pl.loop(0, even_cnt, step=2)
    def _(c):
        dd0 = pltpu.async_copy(
            dstc_hbm.at[pl.ds(r * RCAP + c * 128, 128)], didx.at[0], semd0)
        g0 = pltpu.async_copy(
            p_hbm.at[srcb.at[pl.ds(c * 128, 128)]], rows0, sem0)
        dd1 = pltpu.async_copy(
            dstc_hbm.at[pl.ds(r * RCAP + c * 128 + 128, 128)], didx.at[1],
            semd1)
        g1 = pltpu.async_copy(
            p_hbm.at[srcb.at[pl.ds(c * 128 + 128, 128)]], rows1, sem1)
        dd0.wait()
        g0.wait()
        pltpu.sync_copy(rows0, acc.at[didx.at[0]], add=True)
        dd1.wait()
        g1.wait()
        pltpu.sync_copy(rows1, acc.at[didx.at[1]], add=True)

    @pl.when(even_cnt != rows_cnt)
    def _():
        c = even_cnt
        dd0 = pltpu.async_copy(
            dstc_hbm.at[pl.ds(r * RCAP + c * 128, 128)], didx.at[0], semd0)
        g0 = pltpu.async_copy(
            p_hbm.at[srcb.at[pl.ds(c * 128, 128)]], rows0, sem0)
        dd0.wait()
        g0.wait()
        pltpu.sync_copy(rows0, acc.at[didx.at[0]], add=True)

    plsc.subcore_barrier()
    pltpu.sync_copy(acc.at[pl.ds(nbase, STRIPE)],
                    out_hbm.at[cid].at[pl.ds(nbase, STRIPE)])


_agg_call = pl.kernel(
    _agg_body,
    out_type=jax.ShapeDtypeStruct((2, NP, H), jnp.float32),
    mesh=_mesh,
    scratch_types=[
        pltpu.VMEM((RCAP,), jnp.int32),
        pltpu.VMEM((2, 128), jnp.int32),
        pltpu.VMEM((L,), jnp.int32),
        pltpu.VMEM((128, H), jnp.float32),
        pltpu.VMEM((128, H), jnp.float32),
        pltpu.VMEM((32, H), jnp.float32),
        pltpu.VMEM_SHARED((NP, H), jnp.float32),
        pltpu.SemaphoreType.DMA,
        pltpu.SemaphoreType.DMA,
        pltpu.SemaphoreType.DMA,
        pltpu.SemaphoreType.DMA,
    ],
    compiler_params=_sc_params,
    name="sc_gather_segsum",
)


B = 2560
GP = NP // B


def _pre_body(x_ref, cm_ref, wio_ref, wiu_ref, wo1_ref, wu1_ref,
              xo_ref, xu_ref, p1_ref, s1_ref, s0_ref, sc_ref):
    i = pl.program_id(0)
    x = x_ref[...]
    cm = cm_ref[...]
    xo = lax.dot(x[:, :DIM_O], wio_ref[...], precision=_HIGH)
    xu = lax.dot(x[:, DIM_O:], wiu_ref[...], precision=_HIGH)
    xo_ref[...] = xo
    xu_ref[...] = xu
    p1_ref[...] = (lax.dot(xo, wo1_ref[...], precision=_HIGH)
                   + lax.dot(xu, wu1_ref[...], precision=_HIGH))

    @pl.when(i == 0)
    def _():
        s1_ref[...] = jnp.zeros_like(s1_ref)
        s0_ref[...] = jnp.zeros_like(s0_ref)
        sc_ref[...] = jnp.zeros_like(sc_ref)

    s1_ref[...] += jnp.sum(xo * cm, axis=0, keepdims=True)
    s0_ref[...] += jnp.sum(xo * (1.0 - cm), axis=0, keepdims=True)
    sc_ref[...] += jnp.sum(cm, keepdims=True)


_pre_call = pl.pallas_call(
    _pre_body,
    grid=(GP,),
    in_specs=[
        pl.BlockSpec((B, 2 * DIM_O), lambda i: (i, 0)),
        pl.BlockSpec((B, 1), lambda i: (i, 0)),
        pl.BlockSpec((DIM_O, H), lambda i: (0, 0)),
        pl.BlockSpec((DIM_O, H), lambda i: (0, 0)),
        pl.BlockSpec((H, H), lambda i: (0, 0)),
        pl.BlockSpec((H, H), lambda i: (0, 0)),
    ],
    out_specs=[
        pl.BlockSpec((B, H), lambda i: (i, 0)),
        pl.BlockSpec((B, H), lambda i: (i, 0)),
        pl.BlockSpec((B, H), lambda i: (i, 0)),
        pl.BlockSpec((1, H), lambda i: (0, 0)),
        pl.BlockSpec((1, H), lambda i: (0, 0)),
        pl.BlockSpec((1, 1), lambda i: (0, 0)),
    ],
    out_shape=(
        jax.ShapeDtypeStruct((NP, H), jnp.float32),
        jax.ShapeDtypeStruct((NP, H), jnp.float32),
        jax.ShapeDtypeStruct((NP, H), jnp.float32),
        jax.ShapeDtypeStruct((1, H), jnp.float32),
        jax.ShapeDtypeStruct((1, H), jnp.float32),
        jax.ShapeDtypeStruct((1, 1), jnp.float32),
    ),
)


def _t1_body(cpart_ref, cm_ref, ns_ref, mt_ref):
    tot = cpart_ref[0:1, :] + cpart_ref[1:2, :]
    cm = cm_ref[...]
    one = jnp.ones_like(tot)
    zero = jnp.zeros_like(tot)
    ns_ref[...] = jnp.where(tot > 0, one, zero)
    mt_ref[...] = jnp.where((tot == 0) & (cm == 0), one, zero)


_t1_call = pl.pallas_call(
    _t1_body,
    out_shape=(
        jax.ShapeDtypeStruct((1, NP), jnp.int32),
        jax.ShapeDtypeStruct((1, NP), jnp.int32),
    ),
)


def _mid_body(xo_ref, xu_ref, part_ref, cnt_ref, s1_ref, s0_ref, sc_ref,
              wd_ref, wo2_ref, wu2_ref, xuh_ref, p2_ref):
    nc = jnp.maximum(sc_ref[0, 0], 1.0)
    nnc = jnp.maximum(jnp.float32(N) - sc_ref[0, 0], 1.0)
    delta = s1_ref[...] / nc - s0_ref[...] / nnc
    add = lax.dot(delta, wd_ref[...], precision=_HIGH)
    deg = (cnt_ref[0] + cnt_ref[1]).astype(jnp.float32)
    agg = (part_ref[0] + part_ref[1]) / jnp.maximum(deg, 1.0)
    xuh = xu_ref[...] + agg + add
    xuh_ref[...] = xuh
    p2_ref[...] = (lax.dot(xo_ref[...], wo2_ref[...], precision=_HIGH)
                   + lax.dot(xuh, wu2_ref[...], precision=_HIGH))


_mid_call = pl.pallas_call(
    _mid_body,
    grid=(GP,),
    in_specs=[
        pl.BlockSpec((B, H), lambda i: (i, 0)),
        pl.BlockSpec((B, H), lambda i: (i, 0)),
        pl.BlockSpec((2, B, H), lambda i: (0, i, 0)),
        pl.BlockSpec((2, B, 1), lambda i: (0, i, 0)),
        pl.BlockSpec((1, H), lambda i: (0, 0)),
        pl.BlockSpec((1, H), lambda i: (0, 0)),
        pl.BlockSpec((1, 1), lambda i: (0, 0)),
        pl.BlockSpec((H, H), lambda i: (0, 0)),
        pl.BlockSpec((H, H), lambda i: (0, 0)),
        pl.BlockSpec((H, H), lambda i: (0, 0)),
    ],
    out_specs=[
        pl.BlockSpec((B, H), lambda i: (i, 0)),
        pl.BlockSpec((B, H), lambda i: (i, 0)),
    ],
    out_shape=(
        jax.ShapeDtypeStruct((NP, H), jnp.float32),
        jax.ShapeDtypeStruct((NP, H), jnp.float32),
    ),
)


def _post_body(xo_ref, xu_ref, xuh_ref, part_ref, cnt_ref, cm_ref, out_ref):
    deg = (cnt_ref[0] + cnt_ref[1]).astype(jnp.float32)
    final_u = xuh_ref[...] + (part_ref[0] + part_ref[1]) / jnp.maximum(deg, 1.0)
    final_u = jnp.where(cm_ref[...] > 0.0, xu_ref[...], final_u)
    out_ref[...] = jnp.concatenate([xo_ref[...], final_u], axis=1)


_post_call = pl.pallas_call(
    _post_body,
    grid=(GP,),
    in_specs=[
        pl.BlockSpec((B, H), lambda i: (i, 0)),
        pl.BlockSpec((B, H), lambda i: (i, 0)),
        pl.BlockSpec((B, H), lambda i: (i, 0)),
        pl.BlockSpec((2, B, H), lambda i: (0, i, 0)),
        pl.BlockSpec((2, B, 1), lambda i: (0, i, 0)),
        pl.BlockSpec((B, 1), lambda i: (i, 0)),
    ],
    out_specs=pl.BlockSpec((B, 2 * H), lambda i: (i, 0)),
    out_shape=jax.ShapeDtypeStruct((NP, 2 * H), jnp.float32),
)


@jax.jit
def _impl(x, edge_index, central_mask,
          W_in_o, W_in_u, W_o1, W_u1, W_delta, W_o2, W_u2):
    xp = jnp.zeros((NP, DIM_O + DIM_O), x.dtype).at[:N].set(x)
    cm_i = jnp.zeros((NP,), jnp.int32).at[:N].set(central_mask.astype(jnp.int32))
    cm_f = cm_i.astype(jnp.float32).reshape(NP, 1)
    pad = jnp.full((EP - E,), TRASH, jnp.int32)
    src1 = jnp.concatenate([edge_index[0], pad])
    dst1 = jnp.concatenate([edge_index[1], pad])
    zer = jnp.zeros((32, H), jnp.float32)

    sc1, dc1, cnt1, cp1 = _m1_call(src1, dst1, cm_i)
    xo, xu, p1, s1, s0, sc = _pre_call(xp, cm_f, W_in_o, W_in_u, W_o1, W_u1)
    ns, mt2 = _t1_call(cp1, cm_i.reshape(1, NP))
    part1 = _agg_call(p1, sc1, dc1, cnt1, zer)
    sc2, dc2, cnt2, cp2 = _m2_call(src1, dst1, ns.reshape(NP), mt2.reshape(NP))
    xuh, p2 = _mid_call(xo, xu, part1, cp1.reshape(2, NP, 1), s1, s0, sc,
                        W_delta, W_o2, W_u2)
    part2 = _agg_call(p2, sc2, dc2, cnt2, zer)
    out = _post_call(xo, xu, xuh, part2, cp2.reshape(2, NP, 1), cm_f)
    return out[:N]


def kernel(x, edge_index, central_mask,
           W_in_o, W_in_u, W_o1, W_u1, W_delta, W_o2, W_u2):
    return _impl(x, edge_index, central_mask,
                 W_in_o, W_in_u, W_o1, W_u1, W_delta, W_o2, W_u2)
